# baseline (speedup 1.0000x reference)
import jax
import jax.numpy as jnp
from jax.experimental import pallas as pl

BS = 128
CH = 32
L = 3072
RED = 6
N = CH * RED
K = 8
T6 = L // RED
NC = 40


def _leaky(v):
    return jnp.where(v > 0, v, 0.01 * v)


def _im2col(xp, ks, lout):
    return jnp.concatenate([xp[:, j:j + lout] for j in range(ks)], axis=0)


def _reflect_pad(h, p):
    lw = h.shape[1]
    left = jnp.concatenate([h[:, p - i:p - i + 1] for i in range(p)], axis=1)
    right = jnp.concatenate(
        [h[:, lw - 2 - i:lw - 1 - i] for i in range(p)], axis=1)
    return jnp.concatenate([left, h, right], axis=1)


def _feat_kernel(x_ref, w0_ref, b0_ref, w1_ref, b1_ref, w2_ref, b2_ref,
                 hf_ref):
    xb = x_ref[0]
    xp = _reflect_pad(xb, 7)
    w0 = w0_ref[...]
    b0 = b0_ref[...]
    chunks = []
    for c in range(6):
        a = _im2col(xp[:, c * 512:c * 512 + 512 + 14], 15, 512)
        o = jnp.dot(w0, a, preferred_element_type=jnp.float32) + b0
        o = _leaky(o)
        chunks.append(o.reshape(64, 128, 4).mean(-1))
    h1 = jnp.concatenate(chunks, axis=1)
    a1 = _im2col(_reflect_pad(h1, 4), 9, 768)
    o1 = _leaky(jnp.dot(w1_ref[...], a1,
                        preferred_element_type=jnp.float32) + b1_ref[...])
    h2 = o1.reshape(64, 192, 4).mean(-1)
    a2 = _im2col(_reflect_pad(h2, 2), 5, 192)
    o2 = jnp.tanh(jnp.dot(w2_ref[...], a2,
                          preferred_element_type=jnp.float32) + b2_ref[...])
    hf_ref[0] = o2


def _router_kernel(hf_ref, w1_ref, w2_ref, sel_ref):
    h = jnp.zeros((BS, N), jnp.float32)
    for c in range(16):
        h = h + jnp.dot(hf_ref[:, c, :], w1_ref[c],
                        preferred_element_type=jnp.float32)
    h = _leaky(h)
    logits = jnp.dot(h, w2_ref[...], preferred_element_type=jnp.float32)
    iota = jax.lax.broadcasted_iota(jnp.int32, (BS, N), 1)
    active = jnp.ones((BS, N), jnp.bool_)
    sel = jnp.zeros((BS, N), jnp.float32)
    neg = jnp.float32(-1e30)
    for _ in range(K):
        vals = jnp.where(active, logits, neg)
        m = jnp.max(vals, axis=1, keepdims=True)
        cand = vals == m
        idx = jnp.min(jnp.where(cand, iota, N), axis=1, keepdims=True)
        pick = iota == idx
        sel = sel + pick.astype(jnp.float32)
        active = jnp.logical_and(active, jnp.logical_not(pick))
    sel_ref[...] = sel.reshape(BS, CH, RED)


def _head_kernel(x_ref, sel_ref, w3_ref, gb_ref, w4_ref, b4_ref, w5_ref,
                 b5_ref, cw_ref, cb_ref, out_ref):
    xb = x_ref[0]
    selb = sel_ref[0]
    agg = jnp.sum(xb.reshape(CH, RED, T6) * selb[:, :, None], axis=1)
    a3 = _im2col(agg, 15, 498)
    o3 = jnp.dot(w3_ref[...], a3, preferred_element_type=jnp.float32)
    n0 = jnp.sum(selb[0:1, :])
    o3 = _leaky(o3 + n0 * gb_ref[...])
    p3 = o3.reshape(64, 249, 2).max(-1)
    a4 = _im2col(p3, 9, 241)
    o4 = _leaky(jnp.dot(w4_ref[...], a4,
                        preferred_element_type=jnp.float32) + b4_ref[...])
    p4 = o4[:, :240].reshape(128, 120, 2).max(-1)
    a5 = _im2col(p4, 5, 116)
    o5 = jnp.dot(w5_ref[...], a5,
                 preferred_element_type=jnp.float32) + b5_ref[...]
    e = jnp.mean(o5, axis=1, keepdims=True)
    z = jnp.dot(cw_ref[...], e, preferred_element_type=jnp.float32) \
        + cb_ref[...]
    m = jnp.max(z, axis=0, keepdims=True)
    lse = m + jnp.log(jnp.sum(jnp.exp(z - m), axis=0, keepdims=True))
    out_ref[0] = z - lse


def kernel(x, hw0, hb0, hw1, hb1, hw2, hb2, hlw1, hlw2, gW, gb, ew1, eb1,
           ew2, eb2, cw, cb):
    f32 = jnp.float32
    w0 = hw0.transpose(0, 2, 1).reshape(64, 15 * 32)
    w1 = hw1.transpose(0, 2, 1).reshape(64, 9 * 64)
    w2 = hw2.transpose(0, 2, 1).reshape(16, 5 * 64)
    fw1 = hlw1.reshape(N, 16, 192).transpose(1, 2, 0)
    fw2 = hlw2.T
    w3 = gW.transpose(1, 2, 0).reshape(64, 15 * 32)
    w4 = ew1.transpose(0, 2, 1).reshape(128, 9 * 64)
    w5 = ew2.transpose(0, 2, 1).reshape(256, 5 * 128)
    b0 = hb0.reshape(64, 1)
    b1 = hb1.reshape(64, 1)
    b2 = hb2.reshape(16, 1)
    gbc = gb.reshape(64, 1)
    b4 = eb1.reshape(128, 1)
    b5 = eb2.reshape(256, 1)
    cbc = cb.reshape(NC, 1)

    hf = pl.pallas_call(
        _feat_kernel,
        grid=(BS,),
        in_specs=[
            pl.BlockSpec((1, CH, L), lambda b: (b, 0, 0)),
            pl.BlockSpec((64, 480), lambda b: (0, 0)),
            pl.BlockSpec((64, 1), lambda b: (0, 0)),
            pl.BlockSpec((64, 576), lambda b: (0, 0)),
            pl.BlockSpec((64, 1), lambda b: (0, 0)),
            pl.BlockSpec((16, 320), lambda b: (0, 0)),
            pl.BlockSpec((16, 1), lambda b: (0, 0)),
        ],
        out_specs=pl.BlockSpec((1, 16, 192), lambda b: (b, 0, 0)),
        out_shape=jax.ShapeDtypeStruct((BS, 16, 192), f32),
    )(x, w0, b0, w1, b1, w2, b2)

    sel = pl.pallas_call(
        _router_kernel,
        out_shape=jax.ShapeDtypeStruct((BS, CH, RED), f32),
    )(hf, fw1, fw2)

    out = pl.pallas_call(
        _head_kernel,
        grid=(BS,),
        in_specs=[
            pl.BlockSpec((1, CH, L), lambda b: (b, 0, 0)),
            pl.BlockSpec((1, CH, RED), lambda b: (b, 0, 0)),
            pl.BlockSpec((64, 480), lambda b: (0, 0)),
            pl.BlockSpec((64, 1), lambda b: (0, 0)),
            pl.BlockSpec((128, 576), lambda b: (0, 0)),
            pl.BlockSpec((128, 1), lambda b: (0, 0)),
            pl.BlockSpec((256, 640), lambda b: (0, 0)),
            pl.BlockSpec((256, 1), lambda b: (0, 0)),
            pl.BlockSpec((NC, 256), lambda b: (0, 0)),
            pl.BlockSpec((NC, 1), lambda b: (0, 0)),
        ],
        out_specs=pl.BlockSpec((1, NC, 1), lambda b: (b, 0, 0)),
        out_shape=jax.ShapeDtypeStruct((BS, NC, 1), f32),
    )(x, sel, w3, gbc, w4, b4, w5, b5, cw, cbc)

    return out.reshape(BS, NC)


# speedup vs baseline: 8.5087x; 8.5087x over previous
import jax
import jax.numpy as jnp
from jax.experimental import pallas as pl

BS = 128
CH = 32
L = 3072
RED = 6
N = CH * RED
K = 8
T6 = L // RED
NC = 40

BF = jnp.bfloat16
F32 = jnp.float32


def _leaky(v):
    return jnp.maximum(v, v * v.dtype.type(0.01))


def _im2col(xp, ks, lout):
    return jnp.concatenate([xp[:, j:j + lout] for j in range(ks)], axis=0)


def _edge_cols(h, cols):
    return jnp.concatenate([h[:, c:c + 1] for c in cols], axis=1)


def _bdot(a, b):
    return jnp.dot(a.astype(BF), b, preferred_element_type=F32)


def _fdot(a, b):
    return jnp.dot(a, b, preferred_element_type=F32)


def _feat_kernel(x_ref, w0_ref, b0_ref, p4a_ref, w1_ref, b1_ref, p4b_ref,
                 w2_ref, b2_ref, hf_ref):
    for i in range(4):
        _feat_row(i, x_ref, w0_ref, b0_ref, p4a_ref, w1_ref, b1_ref,
                  p4b_ref, w2_ref, b2_ref, hf_ref)


def _feat_row(i, x_ref, w0_ref, b0_ref, p4a_ref, w1_ref, b1_ref, p4b_ref,
              w2_ref, b2_ref, hf_ref):
    xb = x_ref[i]
    w0 = w0_ref[...]
    b0 = b0_ref[...]
    p4a = p4a_ref[...]
    chunks = []
    for c in range(6):
        if c == 0:
            o = _leaky(_bdot(w0, _im2col(xb[:, 0:519], 15, 505)) + b0)
            xe = jnp.concatenate(
                [_edge_cols(xb, [7, 6, 5, 4, 3, 2, 1]), xb[:, 0:14]], axis=1)
            oe = _leaky(_bdot(w0, _im2col(xe, 15, 7)) + b0)
            pc = _bdot(o, p4a[7:512]) + _bdot(oe, p4a[0:7])
        elif c == 5:
            o = _leaky(_bdot(w0, _im2col(xb[:, 2553:3072], 15, 505)) + b0)
            xe = jnp.concatenate(
                [xb[:, 3058:3072],
                 _edge_cols(xb, [3070, 3069, 3068, 3067, 3066, 3065, 3064])],
                axis=1)
            oe = _leaky(_bdot(w0, _im2col(xe, 15, 7)) + b0)
            pc = _bdot(o, p4a[0:505]) + _bdot(oe, p4a[505:512])
        else:
            o = _leaky(_bdot(
                w0, _im2col(xb[:, c * 512 - 7:c * 512 + 519], 15, 512)) + b0)
            pc = _bdot(o, p4a)
        chunks.append(pc)
    h1 = jnp.concatenate(chunks, axis=1).astype(BF)
    w1 = w1_ref[...]
    b1 = b1_ref[...]
    p4b = p4b_ref[...]
    o1m = _leaky(_bdot(w1, _im2col(h1, 9, 760)) + b1)
    xl = jnp.concatenate([_edge_cols(h1, [4, 3, 2, 1]), h1[:, 0:8]], axis=1)
    xr = jnp.concatenate([h1[:, 760:768],
                          _edge_cols(h1, [766, 765, 764, 763])], axis=1)
    o1l = _leaky(_bdot(w1, _im2col(xl, 9, 4)) + b1)
    o1r = _leaky(_bdot(w1, _im2col(xr, 9, 4)) + b1)
    h2 = (_bdot(o1m, p4b[4:764, :]) + _bdot(o1l, p4b[0:4, :])
          + _bdot(o1r, p4b[764:768, :])).astype(BF)
    w2 = w2_ref[...]
    b2 = b2_ref[...]
    o2m = _fdot(w2, _im2col(h2, 5, 188)) + b2
    x2l = _edge_cols(h2, [2, 1, 0, 1, 2, 3])
    x2r = _edge_cols(h2, [188, 189, 190, 191, 190, 189])
    o2l = _fdot(w2, _im2col(x2l, 5, 2)) + b2
    o2r = _fdot(w2, _im2col(x2r, 5, 2)) + b2
    hf_ref[i] = jnp.tanh(jnp.concatenate([o2l, o2m, o2r], axis=1))


def _router_kernel(hf_ref, w1_ref, w2_ref, sel_ref):
    h = jnp.zeros((BS, N), F32)
    for c in range(16):
        h = h + _fdot(hf_ref[:, c, :], w1_ref[c])
    h = _leaky(h)
    logits = _fdot(h, w2_ref[...])
    iota = jax.lax.broadcasted_iota(jnp.int32, (BS, N), 1)
    active = jnp.ones((BS, N), jnp.bool_)
    sel = jnp.zeros((BS, N), F32)
    neg = jnp.float32(-1e30)
    for _ in range(K):
        vals = jnp.where(active, logits, neg)
        m = jnp.max(vals, axis=1, keepdims=True)
        cand = vals == m
        idx = jnp.min(jnp.where(cand, iota, N), axis=1, keepdims=True)
        pick = iota == idx
        sel = sel + pick.astype(F32)
        active = jnp.logical_and(active, jnp.logical_not(pick))
    sel_ref[...] = sel.astype(BF).reshape(BS, CH, RED)


def _head_kernel(x_ref, sel_ref, s2a_ref, s2b_ref, w3_ref, gb_ref, w4_ref,
                 b4_ref, w5_ref, b5_ref, cw_ref, cb_ref, out_ref):
    for i in range(4):
        _head_row(i, x_ref, sel_ref, s2a_ref, s2b_ref, w3_ref, gb_ref,
                  w4_ref, b4_ref, w5_ref, b5_ref, cw_ref, cb_ref, out_ref)


def _head_row(i, x_ref, sel_ref, s2a_ref, s2b_ref, w3_ref, gb_ref, w4_ref,
              b4_ref, w5_ref, b5_ref, cw_ref, cb_ref, out_ref):
    xb = x_ref[i]
    selb = sel_ref[i]
    agg = selb[:, 0:1] * xb[:, 0:T6]
    for r in range(1, RED):
        agg = agg + selb[:, r:r + 1] * xb[:, r * T6:(r + 1) * T6]
    o3 = _bdot(w3_ref[...], _im2col(agg, 15, 498))
    n0 = jnp.sum(selb[0:1, :])
    o3 = _leaky(o3 + n0 * gb_ref[...])
    m3 = jnp.maximum(o3[:, 0:497], o3[:, 1:498])
    p3 = _bdot(m3, s2a_ref[...]).astype(BF)
    o4 = _leaky(_bdot(w4_ref[...], _im2col(p3, 9, 241)) + b4_ref[...])
    m4 = jnp.maximum(o4[:, 0:239], o4[:, 1:240])
    p4 = _bdot(m4, s2b_ref[...]).astype(BF)
    o5 = _fdot(w5_ref[...], _im2col(p4, 5, 116)) + b5_ref[...]
    e = jnp.mean(o5, axis=1, keepdims=True)
    z = _fdot(cw_ref[...], e) + cb_ref[...]
    m = jnp.max(z, axis=0, keepdims=True)
    lse = m + jnp.log(jnp.sum(jnp.exp(z - m), axis=0, keepdims=True))
    out_ref[i] = z - lse


def kernel(x, hw0, hb0, hw1, hb1, hw2, hb2, hlw1, hlw2, gW, gb, ew1, eb1,
           ew2, eb2, cw, cb):
    xbf = x.astype(BF)
    w0 = hw0.transpose(0, 2, 1).reshape(64, 15 * 32).astype(BF)
    w1 = hw1.transpose(0, 2, 1).reshape(64, 9 * 64).astype(BF)
    w2 = hw2.transpose(0, 2, 1).reshape(16, 5 * 64).astype(BF)
    fw1 = hlw1.reshape(N, 16, 192).transpose(1, 2, 0)
    fw2 = hlw2.T
    w3 = gW.transpose(1, 2, 0).reshape(64, 15 * 32).astype(BF)
    w4 = ew1.transpose(0, 2, 1).reshape(128, 9 * 64).astype(BF)
    w5 = ew2.transpose(0, 2, 1).reshape(256, 5 * 128).astype(BF)
    b0 = hb0.reshape(64, 1).astype(BF)
    b1 = hb1.reshape(64, 1).astype(BF)
    b2 = hb2.reshape(16, 1)
    gbc = gb.reshape(64, 1).astype(BF)
    b4 = eb1.reshape(128, 1).astype(BF)
    b5 = eb2.reshape(256, 1)
    cbc = cb.reshape(NC, 1)
    p4a = ((jnp.equal(jnp.arange(512)[:, None] // 4,
                      jnp.arange(128)[None, :])).astype(F32) * 0.25).astype(BF)
    p4b = ((jnp.equal(jnp.arange(768)[:, None] // 4,
                      jnp.arange(192)[None, :])).astype(F32) * 0.25).astype(BF)
    s2a = (jnp.equal(jnp.arange(497)[:, None],
                     2 * jnp.arange(249)[None, :])).astype(BF)
    s2b = (jnp.equal(jnp.arange(239)[:, None],
                     2 * jnp.arange(120)[None, :])).astype(BF)

    hf = pl.pallas_call(
        _feat_kernel,
        grid=(BS // 4,),
        in_specs=[
            pl.BlockSpec((4, CH, L), lambda b: (b, 0, 0)),
            pl.BlockSpec((64, 480), lambda b: (0, 0)),
            pl.BlockSpec((64, 1), lambda b: (0, 0)),
            pl.BlockSpec((512, 128), lambda b: (0, 0)),
            pl.BlockSpec((64, 576), lambda b: (0, 0)),
            pl.BlockSpec((64, 1), lambda b: (0, 0)),
            pl.BlockSpec((768, 192), lambda b: (0, 0)),
            pl.BlockSpec((16, 320), lambda b: (0, 0)),
            pl.BlockSpec((16, 1), lambda b: (0, 0)),
        ],
        out_specs=pl.BlockSpec((4, 16, 192), lambda b: (b, 0, 0)),
        out_shape=jax.ShapeDtypeStruct((BS, 16, 192), F32),
    )(xbf, w0, b0, p4a, w1, b1, p4b, w2, b2)

    sel = pl.pallas_call(
        _router_kernel,
        out_shape=jax.ShapeDtypeStruct((BS, CH, RED), BF),
    )(hf, fw1, fw2)

    out = pl.pallas_call(
        _head_kernel,
        grid=(BS // 4,),
        in_specs=[
            pl.BlockSpec((4, CH, L), lambda b: (b, 0, 0)),
            pl.BlockSpec((4, CH, RED), lambda b: (b, 0, 0)),
            pl.BlockSpec((497, 249), lambda b: (0, 0)),
            pl.BlockSpec((239, 120), lambda b: (0, 0)),
            pl.BlockSpec((64, 480), lambda b: (0, 0)),
            pl.BlockSpec((64, 1), lambda b: (0, 0)),
            pl.BlockSpec((128, 576), lambda b: (0, 0)),
            pl.BlockSpec((128, 1), lambda b: (0, 0)),
            pl.BlockSpec((256, 640), lambda b: (0, 0)),
            pl.BlockSpec((256, 1), lambda b: (0, 0)),
            pl.BlockSpec((NC, 256), lambda b: (0, 0)),
            pl.BlockSpec((NC, 1), lambda b: (0, 0)),
        ],
        out_specs=pl.BlockSpec((4, NC, 1), lambda b: (b, 0, 0)),
        out_shape=jax.ShapeDtypeStruct((BS, NC, 1), F32),
    )(xbf, sel, s2a, s2b, w3, gbc, w4, b4, w5, b5, cw, cbc)

    return out.reshape(BS, NC)
